# baseline (speedup 1.0000x reference)
import functools

import jax
import jax.numpy as jnp
from jax import lax
from jax.experimental import pallas as pl
from jax.experimental.pallas import tpu as pltpu
from jax.experimental.pallas import tpu_sc as plsc

D = 128
S = 2048
B = 4
NC = 2
NS = 16
NW = NC * NS
PW = S // NW
LANES = 16
VPR = D // LANES


def _emb_body(x_hbm, tok_hbm, pos_hbm, out_hbm,
              idx_v, rows_v, pos_v, sem_p, sem_g, sem_s):
    wid = lax.axis_index("s") * NC + lax.axis_index("c")
    base = wid * PW
    idx_cps = [
        pltpu.async_copy(x_hbm.at[b, pl.ds(base, PW)], idx_v.at[b],
                         sem_s.at[b])
        for b in range(B)
    ]
    cp_pos = pltpu.async_copy(pos_hbm.at[pl.ds(base, PW)], pos_v, sem_p)
    gathers = []
    for b in range(B):
        idx_cps[b].wait()
        gathers.append(
            pltpu.async_copy(tok_hbm.at[idx_v.at[b]], rows_v.at[b],
                             sem_g.at[b]))
    cp_pos.wait()

    def add_span(b, lo, n):
        def add_rows(i, carry):
            for u in range(2):
                r = lo + 2 * i + u
                for j in range(VPR):
                    sl = pl.ds(j * LANES, LANES)
                    plsc.addupdate(rows_v.at[b, r, sl], pos_v[r, sl])
            return carry

        lax.fori_loop(0, n // 2, add_rows, 0)

    h = PW // 2
    stores = []
    for b in range(B):
        gathers[b].wait()
        add_span(b, 0, h)
        stores.append(
            pltpu.async_copy(rows_v.at[b, pl.ds(0, h)],
                             out_hbm.at[b, pl.ds(base, h)],
                             sem_s.at[b]))
        add_span(b, h, h)
        stores.append(
            pltpu.async_copy(rows_v.at[b, pl.ds(h, h)],
                             out_hbm.at[b, pl.ds(base + h, h)],
                             sem_g.at[b]))
    for cp in stores:
        cp.wait()


@jax.jit
def kernel(x, token_table, pos_table):
    mesh = plsc.VectorSubcoreMesh(core_axis_name="c", subcore_axis_name="s")
    run = functools.partial(
        pl.kernel,
        mesh=mesh,
        out_type=jax.ShapeDtypeStruct((B, S, D), jnp.float32),
        scratch_types=[
            pltpu.VMEM((B, PW), jnp.int32),
            pltpu.VMEM((B, PW, D), jnp.float32),
            pltpu.VMEM((PW, D), jnp.float32),
            pltpu.SemaphoreType.DMA,
            pltpu.SemaphoreType.DMA((B,)),
            pltpu.SemaphoreType.DMA((B,)),
        ],
    )(_emb_body)
    return run(x.astype(jnp.int32), token_table, pos_table)


# speedup vs baseline: 1.0227x; 1.0227x over previous
import functools

import jax
import jax.numpy as jnp
from jax import lax
from jax.experimental import pallas as pl
from jax.experimental.pallas import tpu as pltpu
from jax.experimental.pallas import tpu_sc as plsc

D = 128
S = 2048
B = 4
NC = 2
NS = 16
NW = NC * NS
PW = S // NW
LANES = 16
VPR = D // LANES


def _emb_body(x_hbm, tok_hbm, pos_hbm, out_hbm,
              idx_v, rows_v, pos_v, sem_p, sem_g, sem_s):
    wid = lax.axis_index("s") * NC + lax.axis_index("c")
    base = wid * PW
    idx_cps = [
        pltpu.async_copy(x_hbm.at[b, pl.ds(base, PW)], idx_v.at[b],
                         sem_s.at[b])
        for b in range(B)
    ]
    cp_pos = pltpu.async_copy(pos_hbm.at[pl.ds(base, PW)], pos_v, sem_p)
    gathers = []
    for b in range(B):
        idx_cps[b].wait()
        gathers.append(
            pltpu.async_copy(tok_hbm.at[idx_v.at[b]], rows_v.at[b],
                             sem_g.at[b]))
    cp_pos.wait()

    def add_span(b, lo, n):
        @plsc.parallel_loop(0, n, step=1, unroll=2)
        def add_rows(i):
            r = lo + i
            for j in range(VPR):
                sl = pl.ds(j * LANES, LANES)
                plsc.addupdate(rows_v.at[b, r, sl], pos_v[r, sl])

    stores = []
    for b in range(B):
        gathers[b].wait()
        if b < B - 1:
            add_span(b, 0, PW)
            stores.append(
                pltpu.async_copy(rows_v.at[b],
                                 out_hbm.at[b, pl.ds(base, PW)],
                                 sem_s.at[b]))
        else:
            h = PW // 2
            add_span(b, 0, h)
            stores.append(
                pltpu.async_copy(rows_v.at[b, pl.ds(0, h)],
                                 out_hbm.at[b, pl.ds(base, h)],
                                 sem_s.at[b]))
            add_span(b, h, h)
            stores.append(
                pltpu.async_copy(rows_v.at[b, pl.ds(h, h)],
                                 out_hbm.at[b, pl.ds(base + h, h)],
                                 sem_p))
    for cp in stores:
        cp.wait()


@jax.jit
def kernel(x, token_table, pos_table):
    mesh = plsc.VectorSubcoreMesh(core_axis_name="c", subcore_axis_name="s")
    run = functools.partial(
        pl.kernel,
        mesh=mesh,
        out_type=jax.ShapeDtypeStruct((B, S, D), jnp.float32),
        scratch_types=[
            pltpu.VMEM((B, PW), jnp.int32),
            pltpu.VMEM((B, PW, D), jnp.float32),
            pltpu.VMEM((PW, D), jnp.float32),
            pltpu.SemaphoreType.DMA,
            pltpu.SemaphoreType.DMA((B,)),
            pltpu.SemaphoreType.DMA((B,)),
        ],
    )(_emb_body)
    return run(x.astype(jnp.int32), token_table, pos_table)


# speedup vs baseline: 1.0316x; 1.0087x over previous
import functools

import jax
import jax.numpy as jnp
from jax import lax
from jax.experimental import pallas as pl
from jax.experimental.pallas import tpu as pltpu
from jax.experimental.pallas import tpu_sc as plsc

D = 128
S = 2048
B = 4
NC = 2
NS = 16
NW = NC * NS
PW = S // NW
LANES = 16
VPR = D // LANES


def _emb_body(x_hbm, tok_hbm, pos_hbm, out_hbm,
              idx_v, rows_v, pos_v, sem_p, sem_g, sem_s):
    wid = lax.axis_index("s") * NC + lax.axis_index("c")
    base = wid * PW
    idx_cps = [
        pltpu.async_copy(x_hbm.at[b, pl.ds(base, PW)], idx_v.at[b],
                         sem_s.at[b])
        for b in range(B)
    ]
    cp_pos = pltpu.async_copy(pos_hbm.at[pl.ds(base, PW)], pos_v, sem_p)
    gathers = []
    for b in range(B):
        idx_cps[b].wait()
        gathers.append(
            pltpu.async_copy(tok_hbm.at[idx_v.at[b]], rows_v.at[b],
                             sem_g.at[b]))
    cp_pos.wait()

    def add_span(b, lo, n):
        @plsc.parallel_loop(0, n, step=1, unroll=1)
        def add_rows(i):
            r = lo + i
            for j in range(VPR):
                sl = pl.ds(j * LANES, LANES)
                plsc.addupdate(rows_v.at[b, r, sl], pos_v[r, sl])

    stores = []
    for b in range(B):
        gathers[b].wait()
        if b < B - 1:
            add_span(b, 0, PW)
            stores.append(
                pltpu.async_copy(rows_v.at[b],
                                 out_hbm.at[b, pl.ds(base, PW)],
                                 sem_s.at[b]))
        else:
            h = PW // 2
            add_span(b, 0, h)
            stores.append(
                pltpu.async_copy(rows_v.at[b, pl.ds(0, h)],
                                 out_hbm.at[b, pl.ds(base, h)],
                                 sem_s.at[b]))
            add_span(b, h, h)
            stores.append(
                pltpu.async_copy(rows_v.at[b, pl.ds(h, h)],
                                 out_hbm.at[b, pl.ds(base + h, h)],
                                 sem_p))
    for cp in stores:
        cp.wait()


@jax.jit
def kernel(x, token_table, pos_table):
    mesh = plsc.VectorSubcoreMesh(core_axis_name="c", subcore_axis_name="s")
    run = functools.partial(
        pl.kernel,
        mesh=mesh,
        out_type=jax.ShapeDtypeStruct((B, S, D), jnp.float32),
        scratch_types=[
            pltpu.VMEM((B, PW), jnp.int32),
            pltpu.VMEM((B, PW, D), jnp.float32),
            pltpu.VMEM((PW, D), jnp.float32),
            pltpu.SemaphoreType.DMA,
            pltpu.SemaphoreType.DMA((B,)),
            pltpu.SemaphoreType.DMA((B,)),
        ],
    )(_emb_body)
    return run(x.astype(jnp.int32), token_table, pos_table)
